# baseline (device time: 103324 ns/iter reference)
import jax
import jax.numpy as jnp
from jax import lax
from jax.experimental import pallas as pl
from jax.experimental.pallas import tpu as pltpu

N_DEV = 4


def kernel(A, B):
    m, _ = A.shape
    _, n = B.shape
    chunk = m // N_DEV

    def body(
        a_ref,
        b_ref,
        out_ref,
        acc_ref,
        rs_send,
        rs_recv,
        ag_own,
        ag_recv,
        rs_send_sems,
        rs_recv_sems,
        ag_send_sems,
        ag_recv_sems,
    ):
        my = lax.axis_index("i")
        left = (my - 1) % N_DEV
        right = (my + 1) % N_DEV

        barrier_sem = pltpu.get_barrier_semaphore()
        for nbr in (left, right):
            pl.semaphore_signal(
                barrier_sem,
                inc=1,
                device_id=(nbr,),
                device_id_type=pl.DeviceIdType.MESH,
            )
        pl.semaphore_wait(barrier_sem, 2)

        a = a_ref[...].astype(jnp.bfloat16)
        b = b_ref[...].astype(jnp.bfloat16)
        acc_ref[...] = jnp.dot(a, b, preferred_element_type=jnp.float32)

        for s in range(N_DEV - 1):
            c_send = (my - s) % N_DEV
            rs_send[s, :, :] = acc_ref[pl.ds(c_send * chunk, chunk), :].astype(
                jnp.bfloat16
            )
            rdma = pltpu.make_async_remote_copy(
                src_ref=rs_send.at[s],
                dst_ref=rs_recv.at[s],
                send_sem=rs_send_sems.at[s],
                recv_sem=rs_recv_sems.at[s],
                device_id=(right,),
                device_id_type=pl.DeviceIdType.MESH,
            )
            rdma.start()
            rdma.wait()
            c_recv = (my - s - 1) % N_DEV
            acc_ref[pl.ds(c_recv * chunk, chunk), :] = acc_ref[
                pl.ds(c_recv * chunk, chunk), :
            ] + rs_recv[s, :, :].astype(jnp.float32)

        own = (my + 1) % N_DEV
        z = acc_ref[pl.ds(own * chunk, chunk), :]
        silu = z / (1.0 + jnp.exp(-z))
        out_ref[pl.ds(own * chunk, chunk), :] = silu
        ag_own[...] = silu.astype(jnp.bfloat16)

        for s in range(N_DEV - 1):
            src = ag_own if s == 0 else ag_recv.at[s - 1]
            rdma = pltpu.make_async_remote_copy(
                src_ref=src,
                dst_ref=ag_recv.at[s],
                send_sem=ag_send_sems.at[s],
                recv_sem=ag_recv_sems.at[s],
                device_id=(right,),
                device_id_type=pl.DeviceIdType.MESH,
            )
            rdma.start()
            rdma.wait()
            c = (my - s) % N_DEV
            out_ref[pl.ds(c * chunk, chunk), :] = ag_recv[s, :, :].astype(
                jnp.float32
            )

    return pl.pallas_call(
        body,
        out_shape=jax.ShapeDtypeStruct((m, n), jnp.float32),
        in_specs=[
            pl.BlockSpec(memory_space=pltpu.VMEM),
            pl.BlockSpec(memory_space=pltpu.VMEM),
        ],
        out_specs=pl.BlockSpec(memory_space=pltpu.VMEM),
        scratch_shapes=[
            pltpu.VMEM((m, n), jnp.float32),
            pltpu.VMEM((N_DEV - 1, chunk, n), jnp.bfloat16),
            pltpu.VMEM((N_DEV - 1, chunk, n), jnp.bfloat16),
            pltpu.VMEM((chunk, n), jnp.bfloat16),
            pltpu.VMEM((N_DEV - 1, chunk, n), jnp.bfloat16),
            pltpu.SemaphoreType.DMA((N_DEV - 1,)),
            pltpu.SemaphoreType.DMA((N_DEV - 1,)),
            pltpu.SemaphoreType.DMA((N_DEV - 1,)),
            pltpu.SemaphoreType.DMA((N_DEV - 1,)),
        ],
        compiler_params=pltpu.CompilerParams(collective_id=0),
    )(A, B)


# device time: 62735 ns/iter; 1.6470x vs baseline; 1.6470x over previous
import jax
import jax.numpy as jnp
from jax import lax
from jax.experimental import pallas as pl
from jax.experimental.pallas import tpu as pltpu

N_DEV = 4


def kernel(A, B):
    m, _ = A.shape
    _, n = B.shape
    chunk = m // N_DEV
    half = n // 2

    def body(
        a_ref,
        b_ref,
        out_ref,
        acc_ref,
        b_bf,
        rs_send_r, rs_recv_r, rs_send_l, rs_recv_l,
        ag_own_r, ag_recv_r, ag_own_l, ag_recv_l,
        rs_ssem_r, rs_rsem_r, rs_ssem_l, rs_rsem_l,
        ag_ssem_r, ag_rsem_r, ag_ssem_l, ag_rsem_l,
    ):
        my = lax.axis_index("i")
        left = (my - 1) % N_DEV
        right = (my + 1) % N_DEV

        barrier_sem = pltpu.get_barrier_semaphore()
        for nbr in (left, right):
            pl.semaphore_signal(
                barrier_sem,
                inc=1,
                device_id=(nbr,),
                device_id_type=pl.DeviceIdType.MESH,
            )
        pl.semaphore_wait(barrier_sem, 2)

        b_bf[...] = b_ref[...].astype(jnp.bfloat16)

        def compute_chunk(c):
            a = a_ref[pl.ds(c * chunk, chunk), :].astype(jnp.bfloat16)
            acc_ref[pl.ds(c * chunk, chunk), :] = jnp.dot(
                a, b_bf[...], preferred_element_type=jnp.float32
            )

        def rdma_pair(s, send_r, recv_r, ssem_r, rsem_r,
                      send_l, recv_l, ssem_l, rsem_l):
            r = pltpu.make_async_remote_copy(
                src_ref=send_r, dst_ref=recv_r.at[s],
                send_sem=ssem_r.at[s], recv_sem=rsem_r.at[s],
                device_id=(right,), device_id_type=pl.DeviceIdType.MESH,
            )
            l = pltpu.make_async_remote_copy(
                src_ref=send_l, dst_ref=recv_l.at[s],
                send_sem=ssem_l.at[s], recv_sem=rsem_l.at[s],
                device_id=(left,), device_id_type=pl.DeviceIdType.MESH,
            )
            r.start()
            l.start()
            return r, l

        rs_pairs = []
        compute_chunk(my)
        for s in range(N_DEV - 1):
            c_r = (my - s) % N_DEV
            c_l = (my + s) % N_DEV
            rs_send_r[s, :, :] = acc_ref[
                pl.ds(c_r * chunk, chunk), pl.ds(0, half)
            ].astype(jnp.bfloat16)
            rs_send_l[s, :, :] = acc_ref[
                pl.ds(c_l * chunk, chunk), pl.ds(half, half)
            ].astype(jnp.bfloat16)
            rs_pairs.append(
                rdma_pair(s, rs_send_r.at[s], rs_recv_r, rs_ssem_r, rs_rsem_r,
                          rs_send_l.at[s], rs_recv_l, rs_ssem_l, rs_rsem_l)
            )
            if s == 0:
                compute_chunk((my - 1) % N_DEV)
                compute_chunk((my + 1) % N_DEV)
            elif s == 1:
                compute_chunk((my + 2) % N_DEV)
            r, l = rs_pairs[s]
            cr_in = (my - s - 1) % N_DEV
            cl_in = (my + s + 1) % N_DEV
            r.wait_recv()
            acc_ref[pl.ds(cr_in * chunk, chunk), pl.ds(0, half)] = acc_ref[
                pl.ds(cr_in * chunk, chunk), pl.ds(0, half)
            ] + rs_recv_r[s, :, :].astype(jnp.float32)
            l.wait_recv()
            acc_ref[pl.ds(cl_in * chunk, chunk), pl.ds(half, half)] = acc_ref[
                pl.ds(cl_in * chunk, chunk), pl.ds(half, half)
            ] + rs_recv_l[s, :, :].astype(jnp.float32)

        own_r = (my + 1) % N_DEV
        own_l = (my - 1) % N_DEV
        z = acc_ref[pl.ds(own_r * chunk, chunk), pl.ds(0, half)]
        silu = z / (1.0 + jnp.exp(-z))
        out_ref[pl.ds(own_r * chunk, chunk), pl.ds(0, half)] = silu
        ag_own_r[...] = silu.astype(jnp.bfloat16)
        z = acc_ref[pl.ds(own_l * chunk, chunk), pl.ds(half, half)]
        silu = z / (1.0 + jnp.exp(-z))
        out_ref[pl.ds(own_l * chunk, chunk), pl.ds(half, half)] = silu
        ag_own_l[...] = silu.astype(jnp.bfloat16)

        ag_pairs = []
        for s in range(N_DEV - 1):
            src_r = ag_own_r if s == 0 else ag_recv_r.at[s - 1]
            src_l = ag_own_l if s == 0 else ag_recv_l.at[s - 1]
            ag_pairs.append(
                rdma_pair(s, src_r, ag_recv_r, ag_ssem_r, ag_rsem_r,
                          src_l, ag_recv_l, ag_ssem_l, ag_rsem_l)
            )
            r, l = ag_pairs[s]
            cr_in = (my - s) % N_DEV
            cl_in = (my + s) % N_DEV
            r.wait_recv()
            out_ref[pl.ds(cr_in * chunk, chunk), pl.ds(0, half)] = ag_recv_r[
                s, :, :
            ].astype(jnp.float32)
            l.wait_recv()
            out_ref[pl.ds(cl_in * chunk, chunk), pl.ds(half, half)] = ag_recv_l[
                s, :, :
            ].astype(jnp.float32)

        for r, l in rs_pairs + ag_pairs:
            r.wait_send()
            l.wait_send()

    nhop = N_DEV - 1
    return pl.pallas_call(
        body,
        out_shape=jax.ShapeDtypeStruct((m, n), jnp.float32),
        in_specs=[
            pl.BlockSpec(memory_space=pltpu.VMEM),
            pl.BlockSpec(memory_space=pltpu.VMEM),
        ],
        out_specs=pl.BlockSpec(memory_space=pltpu.VMEM),
        scratch_shapes=[
            pltpu.VMEM((m, n), jnp.float32),
            pltpu.VMEM(B.shape, jnp.bfloat16),
            pltpu.VMEM((nhop, chunk, half), jnp.bfloat16),
            pltpu.VMEM((nhop, chunk, half), jnp.bfloat16),
            pltpu.VMEM((nhop, chunk, half), jnp.bfloat16),
            pltpu.VMEM((nhop, chunk, half), jnp.bfloat16),
            pltpu.VMEM((chunk, half), jnp.bfloat16),
            pltpu.VMEM((nhop, chunk, half), jnp.bfloat16),
            pltpu.VMEM((chunk, half), jnp.bfloat16),
            pltpu.VMEM((nhop, chunk, half), jnp.bfloat16),
            pltpu.SemaphoreType.DMA((nhop,)),
            pltpu.SemaphoreType.DMA((nhop,)),
            pltpu.SemaphoreType.DMA((nhop,)),
            pltpu.SemaphoreType.DMA((nhop,)),
            pltpu.SemaphoreType.DMA((nhop,)),
            pltpu.SemaphoreType.DMA((nhop,)),
            pltpu.SemaphoreType.DMA((nhop,)),
            pltpu.SemaphoreType.DMA((nhop,)),
        ],
        compiler_params=pltpu.CompilerParams(collective_id=0),
    )(A, B)


# device time: 61449 ns/iter; 1.6815x vs baseline; 1.0209x over previous
import jax
import jax.numpy as jnp
from jax import lax
from jax.experimental import pallas as pl
from jax.experimental.pallas import tpu as pltpu

N_DEV = 4


def kernel(A, B):
    m, _ = A.shape
    _, n = B.shape
    chunk = m // N_DEV
    half = n // 2

    def body(
        a_ref,
        b_ref,
        out_ref,
        b_bf,
        pcL, pcR,
        rs_send_r, rs_recv_r, rs_send_l, rs_recv_l,
        ag_own_r, ag_recv_r, ag_own_l, ag_recv_l,
        rs_ssem_r, rs_rsem_r, rs_ssem_l, rs_rsem_l,
        ag_ssem_r, ag_rsem_r, ag_ssem_l, ag_rsem_l,
    ):
        my = lax.axis_index("i")
        left = (my - 1) % N_DEV
        right = (my + 1) % N_DEV

        barrier_sem = pltpu.get_barrier_semaphore()
        for nbr in (left, right):
            pl.semaphore_signal(
                barrier_sem,
                inc=1,
                device_id=(nbr,),
                device_id_type=pl.DeviceIdType.MESH,
            )
        pl.semaphore_wait(barrier_sem, 2)

        b_bf[...] = b_ref[...].astype(jnp.bfloat16)

        def rdma(src, dst, ssem, rsem, target):
            return pltpu.make_async_remote_copy(
                src_ref=src, dst_ref=dst, send_sem=ssem, recv_sem=rsem,
                device_id=(target,), device_id_type=pl.DeviceIdType.MESH,
            )

        def a_chunk(c):
            return a_ref[pl.ds(c * chunk, chunk), :].astype(jnp.bfloat16)

        a0 = a_chunk(my)
        rs_send_r[0, :, :] = jnp.dot(
            a0, b_bf[:, :half], preferred_element_type=jnp.float32
        ).astype(jnp.bfloat16)
        rs_send_l[0, :, :] = jnp.dot(
            a0, b_bf[:, half:], preferred_element_type=jnp.float32
        ).astype(jnp.bfloat16)

        rs_r = [rdma(rs_send_r.at[s], rs_recv_r.at[s], rs_ssem_r.at[s],
                     rs_rsem_r.at[s], right) for s in range(3)]
        rs_l = [rdma(rs_send_l.at[s], rs_recv_l.at[s], rs_ssem_l.at[s],
                     rs_rsem_l.at[s], left) for s in range(3)]
        rs_r[0].start()
        rs_l[0].start()

        for c in ((my - 1) % N_DEV, (my + 1) % N_DEV, (my + 2) % N_DEV):
            ac = a_chunk(c)
            pcL[pl.ds(c * chunk, chunk), :] = jnp.dot(
                ac, b_bf[:, :half], preferred_element_type=jnp.float32
            ).astype(jnp.bfloat16)
            pcR[pl.ds(c * chunk, chunk), :] = jnp.dot(
                ac, b_bf[:, half:], preferred_element_type=jnp.float32
            ).astype(jnp.bfloat16)

        for s in range(2):
            cr = (my - s - 1) % N_DEV
            rs_r[s].wait_recv()
            rs_send_r[s + 1, :, :] = (
                pcL[pl.ds(cr * chunk, chunk), :] + rs_recv_r[s, :, :]
            )
            rs_r[s + 1].start()
            cl = (my + s + 1) % N_DEV
            rs_l[s].wait_recv()
            rs_send_l[s + 1, :, :] = (
                pcR[pl.ds(cl * chunk, chunk), :] + rs_recv_l[s, :, :]
            )
            rs_l[s + 1].start()

        ag_r = [rdma(ag_own_r if s == 0 else ag_recv_r.at[s - 1],
                     ag_recv_r.at[s], ag_ssem_r.at[s], ag_rsem_r.at[s],
                     right) for s in range(3)]
        ag_l = [rdma(ag_own_l if s == 0 else ag_recv_l.at[s - 1],
                     ag_recv_l.at[s], ag_ssem_l.at[s], ag_rsem_l.at[s],
                     left) for s in range(3)]

        own_r = (my + 1) % N_DEV
        rs_r[2].wait_recv()
        zr = pcL[pl.ds(own_r * chunk, chunk), :].astype(jnp.float32) + (
            rs_recv_r[2, :, :].astype(jnp.float32)
        )
        silu_r = zr / (1.0 + jnp.exp(-zr))
        ag_own_r[...] = silu_r.astype(jnp.bfloat16)
        ag_r[0].start()
        out_ref[pl.ds(own_r * chunk, chunk), pl.ds(0, half)] = silu_r

        own_l = (my - 1) % N_DEV
        rs_l[2].wait_recv()
        zl = pcR[pl.ds(own_l * chunk, chunk), :].astype(jnp.float32) + (
            rs_recv_l[2, :, :].astype(jnp.float32)
        )
        silu_l = zl / (1.0 + jnp.exp(-zl))
        ag_own_l[...] = silu_l.astype(jnp.bfloat16)
        ag_l[0].start()
        out_ref[pl.ds(own_l * chunk, chunk), pl.ds(half, half)] = silu_l

        for s in range(2):
            ag_r[s].wait_recv()
            ag_r[s + 1].start()
            out_ref[
                pl.ds(((my - s) % N_DEV) * chunk, chunk), pl.ds(0, half)
            ] = ag_recv_r[s, :, :].astype(jnp.float32)
            ag_l[s].wait_recv()
            ag_l[s + 1].start()
            out_ref[
                pl.ds(((my + s) % N_DEV) * chunk, chunk), pl.ds(half, half)
            ] = ag_recv_l[s, :, :].astype(jnp.float32)

        ag_r[2].wait_recv()
        out_ref[
            pl.ds(((my - 2) % N_DEV) * chunk, chunk), pl.ds(0, half)
        ] = ag_recv_r[2, :, :].astype(jnp.float32)
        ag_l[2].wait_recv()
        out_ref[
            pl.ds(((my + 2) % N_DEV) * chunk, chunk), pl.ds(half, half)
        ] = ag_recv_l[2, :, :].astype(jnp.float32)

        for op in rs_r + rs_l + ag_r + ag_l:
            op.wait_send()

    nhop = N_DEV - 1
    return pl.pallas_call(
        body,
        out_shape=jax.ShapeDtypeStruct((m, n), jnp.float32),
        in_specs=[
            pl.BlockSpec(memory_space=pltpu.VMEM),
            pl.BlockSpec(memory_space=pltpu.VMEM),
        ],
        out_specs=pl.BlockSpec(memory_space=pltpu.VMEM),
        scratch_shapes=[
            pltpu.VMEM(B.shape, jnp.bfloat16),
            pltpu.VMEM((m, half), jnp.bfloat16),
            pltpu.VMEM((m, half), jnp.bfloat16),
            pltpu.VMEM((nhop, chunk, half), jnp.bfloat16),
            pltpu.VMEM((nhop, chunk, half), jnp.bfloat16),
            pltpu.VMEM((nhop, chunk, half), jnp.bfloat16),
            pltpu.VMEM((nhop, chunk, half), jnp.bfloat16),
            pltpu.VMEM((chunk, half), jnp.bfloat16),
            pltpu.VMEM((nhop, chunk, half), jnp.bfloat16),
            pltpu.VMEM((chunk, half), jnp.bfloat16),
            pltpu.VMEM((nhop, chunk, half), jnp.bfloat16),
            pltpu.SemaphoreType.DMA((nhop,)),
            pltpu.SemaphoreType.DMA((nhop,)),
            pltpu.SemaphoreType.DMA((nhop,)),
            pltpu.SemaphoreType.DMA((nhop,)),
            pltpu.SemaphoreType.DMA((nhop,)),
            pltpu.SemaphoreType.DMA((nhop,)),
            pltpu.SemaphoreType.DMA((nhop,)),
            pltpu.SemaphoreType.DMA((nhop,)),
        ],
        compiler_params=pltpu.CompilerParams(collective_id=0),
    )(A, B)


# device time: 52310 ns/iter; 1.9752x vs baseline; 1.1747x over previous
import jax
import jax.numpy as jnp
from jax import lax
from jax.experimental import pallas as pl
from jax.experimental.pallas import tpu as pltpu

N_DEV = 4
NP = 2


def kernel(A, B):
    m, _ = A.shape
    _, n = B.shape
    chunk = m // N_DEV
    half = n // 2
    piece = chunk // NP

    def body(
        a_ref,
        b_ref,
        out_ref,
        b_bf,
        pcL, pcR,
        rs_send_r, rs_recv_r, rs_send_l, rs_recv_l,
        ag_own_r, ag_recv_r, ag_own_l, ag_recv_l,
        rs_ssem_r, rs_rsem_r, rs_ssem_l, rs_rsem_l,
        ag_ssem_r, ag_rsem_r, ag_ssem_l, ag_rsem_l,
    ):
        my = lax.axis_index("i")
        left = (my - 1) % N_DEV
        right = (my + 1) % N_DEV

        barrier_sem = pltpu.get_barrier_semaphore()
        for nbr in (left, right):
            pl.semaphore_signal(
                barrier_sem,
                inc=1,
                device_id=(nbr,),
                device_id_type=pl.DeviceIdType.MESH,
            )
        pl.semaphore_wait(barrier_sem, 2)

        b_bf[...] = b_ref[...].astype(jnp.bfloat16)

        def rdma(src, dst, ssem, rsem, target):
            return pltpu.make_async_remote_copy(
                src_ref=src, dst_ref=dst, send_sem=ssem, recv_sem=rsem,
                device_id=(target,), device_id_type=pl.DeviceIdType.MESH,
            )

        rs_r = [[rdma(rs_send_r.at[s, p], rs_recv_r.at[s, p],
                      rs_ssem_r.at[s, p], rs_rsem_r.at[s, p], right)
                 for p in range(NP)] for s in range(3)]
        rs_l = [[rdma(rs_send_l.at[s, p], rs_recv_l.at[s, p],
                      rs_ssem_l.at[s, p], rs_rsem_l.at[s, p], left)
                 for p in range(NP)] for s in range(3)]
        ag_r = [[rdma(ag_own_r.at[p] if s == 0 else ag_recv_r.at[s - 1, p],
                      ag_recv_r.at[s, p], ag_ssem_r.at[s, p],
                      ag_rsem_r.at[s, p], right)
                 for p in range(NP)] for s in range(3)]
        ag_l = [[rdma(ag_own_l.at[p] if s == 0 else ag_recv_l.at[s - 1, p],
                      ag_recv_l.at[s, p], ag_ssem_l.at[s, p],
                      ag_rsem_l.at[s, p], left)
                 for p in range(NP)] for s in range(3)]

        def a_rows(c, p):
            return a_ref[
                pl.ds(c * chunk + p * piece, piece), :
            ].astype(jnp.bfloat16)

        for p in range(NP):
            ap = a_rows(my, p)
            rs_send_r[0, p, :, :] = jnp.dot(
                ap, b_bf[:, :half], preferred_element_type=jnp.float32
            ).astype(jnp.bfloat16)
            rs_r[0][p].start()
            rs_send_l[0, p, :, :] = jnp.dot(
                ap, b_bf[:, half:], preferred_element_type=jnp.float32
            ).astype(jnp.bfloat16)
            rs_l[0][p].start()

        for c in ((my - 1) % N_DEV, (my + 1) % N_DEV, (my + 2) % N_DEV):
            ac = a_ref[pl.ds(c * chunk, chunk), :].astype(jnp.bfloat16)
            pcL[pl.ds(c * chunk, chunk), :] = jnp.dot(
                ac, b_bf[:, :half], preferred_element_type=jnp.float32
            ).astype(jnp.bfloat16)
            pcR[pl.ds(c * chunk, chunk), :] = jnp.dot(
                ac, b_bf[:, half:], preferred_element_type=jnp.float32
            ).astype(jnp.bfloat16)

        for s in range(2):
            cr = (my - s - 1) % N_DEV
            cl = (my + s + 1) % N_DEV
            for p in range(NP):
                rs_r[s][p].wait_recv()
                rs_send_r[s + 1, p, :, :] = (
                    pcL[pl.ds(cr * chunk + p * piece, piece), :]
                    + rs_recv_r[s, p, :, :]
                )
                rs_r[s + 1][p].start()
                rs_l[s][p].wait_recv()
                rs_send_l[s + 1, p, :, :] = (
                    pcR[pl.ds(cl * chunk + p * piece, piece), :]
                    + rs_recv_l[s, p, :, :]
                )
                rs_l[s + 1][p].start()

        own_r = (my + 1) % N_DEV
        own_l = (my - 1) % N_DEV
        for p in range(NP):
            rs_r[2][p].wait_recv()
            zr = pcL[
                pl.ds(own_r * chunk + p * piece, piece), :
            ].astype(jnp.float32) + rs_recv_r[2, p, :, :].astype(jnp.float32)
            silu_r = zr / (1.0 + jnp.exp(-zr))
            ag_own_r[p, :, :] = silu_r.astype(jnp.bfloat16)
            ag_r[0][p].start()
            out_ref[
                pl.ds(own_r * chunk + p * piece, piece), pl.ds(0, half)
            ] = silu_r
            rs_l[2][p].wait_recv()
            zl = pcR[
                pl.ds(own_l * chunk + p * piece, piece), :
            ].astype(jnp.float32) + rs_recv_l[2, p, :, :].astype(jnp.float32)
            silu_l = zl / (1.0 + jnp.exp(-zl))
            ag_own_l[p, :, :] = silu_l.astype(jnp.bfloat16)
            ag_l[0][p].start()
            out_ref[
                pl.ds(own_l * chunk + p * piece, piece), pl.ds(half, half)
            ] = silu_l

        for s in range(3):
            cr = (my - s) % N_DEV
            cl = (my + s) % N_DEV
            for p in range(NP):
                ag_r[s][p].wait_recv()
                if s < 2:
                    ag_r[s + 1][p].start()
                out_ref[
                    pl.ds(cr * chunk + p * piece, piece), pl.ds(0, half)
                ] = ag_recv_r[s, p, :, :].astype(jnp.float32)
                ag_l[s][p].wait_recv()
                if s < 2:
                    ag_l[s + 1][p].start()
                out_ref[
                    pl.ds(cl * chunk + p * piece, piece), pl.ds(half, half)
                ] = ag_recv_l[s, p, :, :].astype(jnp.float32)

        for grid in (rs_r, rs_l, ag_r, ag_l):
            for ops in grid:
                for op in ops:
                    op.wait_send()

    nhop = N_DEV - 1
    return pl.pallas_call(
        body,
        out_shape=jax.ShapeDtypeStruct((m, n), jnp.float32),
        in_specs=[
            pl.BlockSpec(memory_space=pltpu.VMEM),
            pl.BlockSpec(memory_space=pltpu.VMEM),
        ],
        out_specs=pl.BlockSpec(memory_space=pltpu.VMEM),
        scratch_shapes=[
            pltpu.VMEM(B.shape, jnp.bfloat16),
            pltpu.VMEM((m, half), jnp.bfloat16),
            pltpu.VMEM((m, half), jnp.bfloat16),
            pltpu.VMEM((nhop, NP, piece, half), jnp.bfloat16),
            pltpu.VMEM((nhop, NP, piece, half), jnp.bfloat16),
            pltpu.VMEM((nhop, NP, piece, half), jnp.bfloat16),
            pltpu.VMEM((nhop, NP, piece, half), jnp.bfloat16),
            pltpu.VMEM((NP, piece, half), jnp.bfloat16),
            pltpu.VMEM((nhop, NP, piece, half), jnp.bfloat16),
            pltpu.VMEM((NP, piece, half), jnp.bfloat16),
            pltpu.VMEM((nhop, NP, piece, half), jnp.bfloat16),
            pltpu.SemaphoreType.DMA((nhop, NP)),
            pltpu.SemaphoreType.DMA((nhop, NP)),
            pltpu.SemaphoreType.DMA((nhop, NP)),
            pltpu.SemaphoreType.DMA((nhop, NP)),
            pltpu.SemaphoreType.DMA((nhop, NP)),
            pltpu.SemaphoreType.DMA((nhop, NP)),
            pltpu.SemaphoreType.DMA((nhop, NP)),
            pltpu.SemaphoreType.DMA((nhop, NP)),
        ],
        compiler_params=pltpu.CompilerParams(collective_id=0),
    )(A, B)
